# baseline (device time: 59886 ns/iter reference)
import jax
import jax.numpy as jnp
from jax import lax
from jax.experimental import pallas as pl
from jax.experimental.pallas import tpu as pltpu

N_DEV = 8


def kernel(A, B):
    m, _ = A.shape
    _, n = B.shape

    def body(a_ref, b_ref, out_ref, comm_ref, send_sems, recv_sems):
        my = lax.axis_index("i")
        left = lax.rem(my + N_DEV - 1, N_DEV)
        right = lax.rem(my + 1, N_DEV)

        barrier_sem = pltpu.get_barrier_semaphore()
        for nbr in (left, right):
            pl.semaphore_signal(
                barrier_sem, inc=1,
                device_id=(nbr,), device_id_type=pl.DeviceIdType.MESH,
            )
        pl.semaphore_wait(barrier_sem, 2)

        a = a_ref[:, :].astype(jnp.bfloat16)
        b = b_ref[:, :].astype(jnp.bfloat16)
        partial = jnp.dot(a, b, preferred_element_type=jnp.float32)
        comm_ref[0, :, :] = partial.astype(jnp.bfloat16)
        out_ref[:, :] = partial

        for h in range(N_DEV - 1):
            rdma = pltpu.make_async_remote_copy(
                src_ref=comm_ref.at[h],
                dst_ref=comm_ref.at[h + 1],
                send_sem=send_sems.at[h],
                recv_sem=recv_sems.at[h],
                device_id=(right,),
                device_id_type=pl.DeviceIdType.MESH,
            )
            rdma.start()
            rdma.wait()
            out_ref[:, :] = out_ref[:, :] + comm_ref[h + 1, :, :].astype(
                jnp.float32
            )

        z = out_ref[:, :]
        out_ref[:, :] = 0.5 * z * (
            1.0 + jnp.tanh(0.7978845608 * (z + 0.044715 * z * z * z))
        )

    return pl.pallas_call(
        body,
        out_shape=jax.ShapeDtypeStruct((m, n), jnp.float32),
        in_specs=[
            pl.BlockSpec(memory_space=pltpu.VMEM),
            pl.BlockSpec(memory_space=pltpu.VMEM),
        ],
        out_specs=pl.BlockSpec(memory_space=pltpu.VMEM),
        scratch_shapes=[
            pltpu.VMEM((N_DEV, m, n), jnp.bfloat16),
            pltpu.SemaphoreType.DMA((N_DEV - 1,)),
            pltpu.SemaphoreType.DMA((N_DEV - 1,)),
        ],
        compiler_params=pltpu.CompilerParams(collective_id=0),
    )(A, B)


# device time: 29707 ns/iter; 2.0159x vs baseline; 2.0159x over previous
import jax
import jax.numpy as jnp
from jax import lax
from jax.experimental import pallas as pl
from jax.experimental.pallas import tpu as pltpu

N_DEV = 8
N_ROUNDS = 3
_PARTNER_XOR = (1, 3, 4)


def kernel(A, B):
    m, _ = A.shape
    _, n = B.shape

    def body(a_ref, b_ref, out_ref, send_ref, recv_ref, send_sems, recv_sems):
        my = lax.axis_index("i")
        partners = [jnp.bitwise_xor(my, x) for x in _PARTNER_XOR]

        barrier_sem = pltpu.get_barrier_semaphore()
        for p in partners:
            pl.semaphore_signal(
                barrier_sem, inc=1,
                device_id=(p,), device_id_type=pl.DeviceIdType.MESH,
            )
        pl.semaphore_wait(barrier_sem, N_ROUNDS)

        a = a_ref[:, :].astype(jnp.bfloat16)
        b = b_ref[:, :].astype(jnp.bfloat16)
        out_ref[:, :] = jnp.dot(a, b, preferred_element_type=jnp.float32)

        for r in range(N_ROUNDS):
            send_ref[r, :, :] = out_ref[:, :].astype(jnp.bfloat16)
            rdma = pltpu.make_async_remote_copy(
                src_ref=send_ref.at[r],
                dst_ref=recv_ref.at[r],
                send_sem=send_sems.at[r],
                recv_sem=recv_sems.at[r],
                device_id=(partners[r],),
                device_id_type=pl.DeviceIdType.MESH,
            )
            rdma.start()
            rdma.wait()
            out_ref[:, :] = out_ref[:, :] + recv_ref[r, :, :].astype(
                jnp.float32
            )

        z = out_ref[:, :]
        out_ref[:, :] = 0.5 * z * (
            1.0 + jnp.tanh(0.7978845608 * (z + 0.044715 * z * z * z))
        )

    return pl.pallas_call(
        body,
        out_shape=jax.ShapeDtypeStruct((m, n), jnp.float32),
        in_specs=[
            pl.BlockSpec(memory_space=pltpu.VMEM),
            pl.BlockSpec(memory_space=pltpu.VMEM),
        ],
        out_specs=pl.BlockSpec(memory_space=pltpu.VMEM),
        scratch_shapes=[
            pltpu.VMEM((N_ROUNDS, m, n), jnp.bfloat16),
            pltpu.VMEM((N_ROUNDS, m, n), jnp.bfloat16),
            pltpu.SemaphoreType.DMA((N_ROUNDS,)),
            pltpu.SemaphoreType.DMA((N_ROUNDS,)),
        ],
        compiler_params=pltpu.CompilerParams(collective_id=0),
    )(A, B)


# device time: 18580 ns/iter; 3.2231x vs baseline; 1.5989x over previous
import jax
import jax.numpy as jnp
from jax import lax
from jax.experimental import pallas as pl
from jax.experimental.pallas import tpu as pltpu

N_DEV = 8


def kernel(A, B):
    m, _ = A.shape
    _, n = B.shape
    mc = m // N_DEV

    def body(a_ref, b_ref, out_ref, part_ref, rs_ref, ag_ref,
             rs_send_sems, rs_recv_sems, ag_send_sems, ag_recv_sems):
        my = lax.axis_index("i")

        barrier_sem = pltpu.get_barrier_semaphore()
        for p in range(N_DEV):

            @pl.when(p != my)
            def _(p=p):
                pl.semaphore_signal(
                    barrier_sem, inc=1,
                    device_id=(p,), device_id_type=pl.DeviceIdType.MESH,
                )

        pl.semaphore_wait(barrier_sem, N_DEV - 1)

        a = a_ref[:, :].astype(jnp.bfloat16)
        b = b_ref[:, :].astype(jnp.bfloat16)
        partial = jnp.dot(a, b, preferred_element_type=jnp.float32)
        part_ref[:, :, :] = partial.reshape(N_DEV, mc, n).astype(jnp.bfloat16)

        rs_rdmas = []
        for p in range(N_DEV):
            rdma = pltpu.make_async_remote_copy(
                src_ref=part_ref.at[p],
                dst_ref=rs_ref.at[my],
                send_sem=rs_send_sems.at[p],
                recv_sem=rs_recv_sems.at[my],
                device_id=(p,),
                device_id_type=pl.DeviceIdType.MESH,
            )
            rs_rdmas.append(rdma)

            @pl.when(p != my)
            def _(rdma=rdma):
                rdma.start()

        rs_ref[my, :, :] = part_ref[my, :, :]

        for s in range(N_DEV):
            recv = pltpu.make_async_remote_copy(
                src_ref=part_ref.at[s],
                dst_ref=rs_ref.at[s],
                send_sem=rs_send_sems.at[s],
                recv_sem=rs_recv_sems.at[s],
                device_id=(s,),
                device_id_type=pl.DeviceIdType.MESH,
            )

            @pl.when(s != my)
            def _(recv=recv):
                recv.wait_recv()

        z = jnp.sum(rs_ref[:, :, :].astype(jnp.float32), axis=0)
        g = 0.5 * z * (
            1.0 + jnp.tanh(0.7978845608 * (z + 0.044715 * z * z * z))
        )
        ag_ref[my, :, :] = g.astype(jnp.bfloat16)

        ag_rdmas = []
        for p in range(N_DEV):
            rdma = pltpu.make_async_remote_copy(
                src_ref=ag_ref.at[my],
                dst_ref=ag_ref.at[my],
                send_sem=ag_send_sems.at[p],
                recv_sem=ag_recv_sems.at[my],
                device_id=(p,),
                device_id_type=pl.DeviceIdType.MESH,
            )
            ag_rdmas.append(rdma)

            @pl.when(p != my)
            def _(rdma=rdma):
                rdma.start()

        for s in range(N_DEV):
            recv = pltpu.make_async_remote_copy(
                src_ref=ag_ref.at[s],
                dst_ref=ag_ref.at[s],
                send_sem=ag_send_sems.at[s],
                recv_sem=ag_recv_sems.at[s],
                device_id=(s,),
                device_id_type=pl.DeviceIdType.MESH,
            )

            @pl.when(s != my)
            def _(recv=recv):
                recv.wait_recv()

        out_ref[:, :] = ag_ref[:, :, :].reshape(m, n).astype(jnp.float32)

        for p in range(N_DEV):

            @pl.when(p != my)
            def _(p=p):
                rs_rdmas[p].wait_send()
                ag_rdmas[p].wait_send()

    return pl.pallas_call(
        body,
        out_shape=jax.ShapeDtypeStruct((m, n), jnp.float32),
        in_specs=[
            pl.BlockSpec(memory_space=pltpu.VMEM),
            pl.BlockSpec(memory_space=pltpu.VMEM),
        ],
        out_specs=pl.BlockSpec(memory_space=pltpu.VMEM),
        scratch_shapes=[
            pltpu.VMEM((N_DEV, mc, n), jnp.bfloat16),
            pltpu.VMEM((N_DEV, mc, n), jnp.bfloat16),
            pltpu.VMEM((N_DEV, mc, n), jnp.bfloat16),
            pltpu.SemaphoreType.DMA((N_DEV,)),
            pltpu.SemaphoreType.DMA((N_DEV,)),
            pltpu.SemaphoreType.DMA((N_DEV,)),
            pltpu.SemaphoreType.DMA((N_DEV,)),
        ],
        compiler_params=pltpu.CompilerParams(collective_id=0),
    )(A, B)


# device time: 17272 ns/iter; 3.4672x vs baseline; 1.0757x over previous
import jax
import jax.numpy as jnp
from jax import lax
from jax.experimental import pallas as pl
from jax.experimental.pallas import tpu as pltpu

N_DEV = 8
_XOR_FAR_FIRST = (6, 2, 5, 7, 1, 3, 4)


def kernel(A, B):
    m, _ = A.shape
    _, n = B.shape
    mc = m // N_DEV

    def body(a_ref, b_ref, out_ref, part_ref, rs_ref,
             rs_send_sems, rs_recv_sems, ag_send_sems, ag_recv_sems):
        my = lax.axis_index("i")

        barrier_sem = pltpu.get_barrier_semaphore()
        for p in range(N_DEV):

            @pl.when(p != my)
            def _(p=p):
                pl.semaphore_signal(
                    barrier_sem, inc=1,
                    device_id=(p,), device_id_type=pl.DeviceIdType.MESH,
                )

        a = a_ref[:, :].astype(jnp.bfloat16)
        b = b_ref[:, :].astype(jnp.bfloat16)
        partial = jnp.dot(a, b, preferred_element_type=jnp.float32)
        part_ref[:, :, :] = partial.reshape(N_DEV, mc, n).astype(jnp.bfloat16)

        pl.semaphore_wait(barrier_sem, N_DEV - 1)

        rs_rdmas = {}
        for x in _XOR_FAR_FIRST:
            p = jnp.bitwise_xor(my, x)
            rdma = pltpu.make_async_remote_copy(
                src_ref=part_ref.at[p],
                dst_ref=rs_ref.at[my],
                send_sem=rs_send_sems.at[x],
                recv_sem=rs_recv_sems.at[my],
                device_id=(p,),
                device_id_type=pl.DeviceIdType.MESH,
            )
            rs_rdmas[x] = rdma
            rdma.start()

        rs_ref[my, :, :] = part_ref[my, :, :]

        for x in _XOR_FAR_FIRST[::-1]:
            s = jnp.bitwise_xor(my, x)
            recv = pltpu.make_async_remote_copy(
                src_ref=part_ref.at[s],
                dst_ref=rs_ref.at[s],
                send_sem=rs_send_sems.at[x],
                recv_sem=rs_recv_sems.at[s],
                device_id=(s,),
                device_id_type=pl.DeviceIdType.MESH,
            )
            recv.wait_recv()

        z = jnp.sum(rs_ref[:, :, :].astype(jnp.float32), axis=0)
        g = 0.5 * z * (
            1.0 + jnp.tanh(0.7978845608 * (z + 0.044715 * z * z * z))
        )
        row0 = my * mc
        out_ref[pl.ds(row0, mc), :] = g.astype(jnp.bfloat16)

        ag_rdmas = {}
        for x in _XOR_FAR_FIRST:
            p = jnp.bitwise_xor(my, x)
            rdma = pltpu.make_async_remote_copy(
                src_ref=out_ref.at[pl.ds(row0, mc)],
                dst_ref=out_ref.at[pl.ds(row0, mc)],
                send_sem=ag_send_sems.at[x],
                recv_sem=ag_recv_sems.at[my],
                device_id=(p,),
                device_id_type=pl.DeviceIdType.MESH,
            )
            ag_rdmas[x] = rdma
            rdma.start()

        for x in _XOR_FAR_FIRST[::-1]:
            s = jnp.bitwise_xor(my, x)
            recv = pltpu.make_async_remote_copy(
                src_ref=out_ref.at[pl.ds(s * mc, mc)],
                dst_ref=out_ref.at[pl.ds(s * mc, mc)],
                send_sem=ag_send_sems.at[x],
                recv_sem=ag_recv_sems.at[s],
                device_id=(s,),
                device_id_type=pl.DeviceIdType.MESH,
            )
            recv.wait_recv()

        for x in _XOR_FAR_FIRST:
            rs_rdmas[x].wait_send()
            ag_rdmas[x].wait_send()

    return pl.pallas_call(
        body,
        out_shape=jax.ShapeDtypeStruct((m, n), jnp.bfloat16),
        in_specs=[
            pl.BlockSpec(memory_space=pltpu.VMEM),
            pl.BlockSpec(memory_space=pltpu.VMEM),
        ],
        out_specs=pl.BlockSpec(memory_space=pltpu.VMEM),
        scratch_shapes=[
            pltpu.VMEM((N_DEV, mc, n), jnp.bfloat16),
            pltpu.VMEM((N_DEV, mc, n), jnp.bfloat16),
            pltpu.SemaphoreType.DMA((N_DEV,)),
            pltpu.SemaphoreType.DMA((N_DEV,)),
            pltpu.SemaphoreType.DMA((N_DEV,)),
            pltpu.SemaphoreType.DMA((N_DEV,)),
        ],
        compiler_params=pltpu.CompilerParams(collective_id=0),
    )(A, B)
